# baseline (speedup 1.0000x reference)
import numpy as np
import jax
import jax.numpy as jnp
from jax.experimental import pallas as pl

_EMB_DIM = 128
_EDGE_CONT = {0: [3, 6, 7, 9, 10, 11, 12, 13], 1: [2, 3, 4, 5, 6, 7, 9, 10, 11, 12, 13], 2: [2, 3, 4, 5, 6, 7, 9, 10, 11, 12, 13], 3: [1, 4, 5, 6, 7, 9, 10, 11, 12, 13], 4: [2, 3, 4, 5, 6, 7, 9, 10, 11, 12, 13], 5: [1, 2, 3, 4, 5, 6, 7, 9, 10, 11, 12, 13], 6: [2, 3, 4, 5, 6, 7, 9, 10, 11, 12, 13], 7: [1, 2, 3, 4, 5, 6, 7, 9, 10, 11, 12, 13], 8: [0, 1, 4, 6, 7, 9, 10, 11, 12, 13]}
_EDGE_DISC_FEATS = {0: [0, 1, 2, 4, 5, 8], 1: [0, 1, 8], 2: [0, 1, 8], 3: [0, 2, 3, 8], 4: [0, 1, 8], 5: [0, 8], 6: [0, 1, 8], 7: [0, 8], 8: [2, 3, 5, 8]}
_COL_RANGES = {0: 4, 1: 6, 2: 6, 3: 8, 4: 15, 5: 2, 8: 9}

_K = 256
_HALF = 128
_BLOCK = 3200

_CONT_LANES = []
for _t in range(9):
    for _c in _EDGE_CONT[_t]:
        _CONT_LANES.append((_c, _t))
def _span_of(t, f):
    feats = _EDGE_DISC_FEATS[t]
    nd = len(feats)
    per, rem = _EMB_DIM // nd, _EMB_DIM % nd
    col = 0
    for i, ff in enumerate(feats):
        dim = per + (1 if i < rem else 0)
        if ff == f:
            return col, dim
        col += dim
    raise KeyError((t, f))


_DISC_GROUPS = []
for _c in [0, 1, 2, 3, 4, 5, 8]:
    bysp = {}
    for _t in range(9):
        if _c in _EDGE_DISC_FEATS[_t]:
            bysp.setdefault(_span_of(_t, _c), []).append(_t)
    for (_lo, _dim), _ts in sorted(bysp.items()):
        _DISC_GROUPS.append((_c, _lo, _dim, _ts))

_DISC_LANES = []
for _c, _lo, _dim, _ts in _DISC_GROUPS:
    for _t in _ts:
        if _c == 8:
            _DISC_LANES.append((8, _t, _t))
        else:
            for _v in range(_COL_RANGES[_c]):
                _DISC_LANES.append((_c, _t, _v))
_NC = len(_CONT_LANES)
_ND = len(_DISC_LANES)
assert _NC + _ND <= _K and _NC <= _HALF

_SS = np.zeros((16, _K + _HALF), np.float32)
_SS[14, :_K] = -1.0
for _i, (_c, _t) in enumerate(_CONT_LANES):
    _SS[8, _i] = 16.0
    _SS[14, _i] = -16.0 * _t
    _SS[_c, _K + _i] = 1.0
for _j, (_c, _t, _v) in enumerate(_DISC_LANES):
    _L = _NC + _j
    _SS[_c, _L] = 1.0 + (16.0 if _c == 8 else 0.0)
    if _c != 8:
        _SS[8, _L] = 16.0
    _SS[14, _L] = -(_v + 16.0 * _t)
    if _L < _HALF:
        _SS[14, _K + _L] = 1.0


def _build_g(params):
    wall = jnp.concatenate([params["W"][str(t)] for t in range(9)], axis=1)
    pieces = [wall.T]
    for c, lo, dim, ts in _DISC_GROUPS:
        if c != 8:
            r = _COL_RANGES[c]
            blk = jnp.concatenate(
                [params["tables"][str(t)][str(c)][:r, :] for t in ts], axis=0)
            pieces.append(jnp.pad(blk, ((0, 0), (lo, _EMB_DIM - lo - dim))))
        else:
            blk = jnp.concatenate(
                [params["tables"][str(t)]["8"][t:t + 1, :] for t in ts], axis=0)
            bias = jnp.stack([params["b"][str(t)] for t in ts], axis=0)
            pieces.append(
                jnp.pad(blk, ((0, 0), (lo, _EMB_DIM - lo - dim))) + bias)
    pieces.append(jnp.zeros((_K - _NC - _ND, _EMB_DIM), jnp.float32))
    return jnp.concatenate(pieces, axis=0).astype(jnp.bfloat16)


_COLMAP = np.zeros(_K, np.int32)
_TVAL = np.full(_K, -1.0, np.float32)
_VVAL = np.zeros(_K, np.float32)
_ISCONT = np.zeros(_K, np.float32)
for _i, (_c, _t) in enumerate(_CONT_LANES):
    _COLMAP[_i], _TVAL[_i], _ISCONT[_i] = _c, _t, 1.0
for _j, (_c, _t, _v) in enumerate(_DISC_LANES):
    _L = _NC + _j
    _COLMAP[_L], _TVAL[_L], _VVAL[_L] = _c, _t, _v


def _body(a_ref, ci_ref, cf_ref, g_ref, o_ref):
    a = a_ref[:, :]
    b = a.shape[0]
    aexp = jnp.take_along_axis(
        a, jnp.broadcast_to(ci_ref[0:1, :], (b, _K)), axis=1)
    t8 = a[:, 8:9]
    tvc = cf_ref[0:1, :]
    vvc = cf_ref[1:2, :]
    icc = cf_ref[2:3, :] != 0.0
    hit = (t8 == tvc) & (icc | (aexp == vvc))
    phi = jnp.where(hit, jnp.where(icc, aexp, 1.0), 0.0).astype(jnp.bfloat16)
    o_ref[:, :] = jnp.dot(phi, g_ref[:, :],
                          preferred_element_type=jnp.float32)


def kernel(edge_attr, params):
    n = edge_attr.shape[0]
    g = _build_g(params)
    grid = n // _BLOCK
    ci = np.zeros((8, _K), np.int32)
    ci[0] = _COLMAP
    cf = np.zeros((8, _K), np.float32)
    cf[0], cf[1], cf[2] = _TVAL, _VVAL, _ISCONT
    return pl.pallas_call(
        _body,
        grid=(grid,),
        in_specs=[
            pl.BlockSpec((_BLOCK, 14), lambda i: (i, 0)),
            pl.BlockSpec((8, _K), lambda i: (0, 0)),
            pl.BlockSpec((8, _K), lambda i: (0, 0)),
            pl.BlockSpec((_K, _EMB_DIM), lambda i: (0, 0)),
        ],
        out_specs=pl.BlockSpec((_BLOCK, _EMB_DIM), lambda i: (i, 0)),
        out_shape=jax.ShapeDtypeStruct((n, _EMB_DIM), jnp.float32),
    )(edge_attr, jnp.asarray(ci), jnp.asarray(cf), g)


# speedup vs baseline: 1.2618x; 1.2618x over previous
import numpy as np
import jax
import jax.numpy as jnp
from jax.experimental import pallas as pl
from jax.experimental.pallas import tpu as pltpu

_EMB_DIM = 128
_EDGE_CONT = {0: [3, 6, 7, 9, 10, 11, 12, 13], 1: [2, 3, 4, 5, 6, 7, 9, 10, 11, 12, 13], 2: [2, 3, 4, 5, 6, 7, 9, 10, 11, 12, 13], 3: [1, 4, 5, 6, 7, 9, 10, 11, 12, 13], 4: [2, 3, 4, 5, 6, 7, 9, 10, 11, 12, 13], 5: [1, 2, 3, 4, 5, 6, 7, 9, 10, 11, 12, 13], 6: [2, 3, 4, 5, 6, 7, 9, 10, 11, 12, 13], 7: [1, 2, 3, 4, 5, 6, 7, 9, 10, 11, 12, 13], 8: [0, 1, 4, 6, 7, 9, 10, 11, 12, 13]}
_EDGE_DISC_FEATS = {0: [0, 1, 2, 4, 5, 8], 1: [0, 1, 8], 2: [0, 1, 8], 3: [0, 2, 3, 8], 4: [0, 1, 8], 5: [0, 8], 6: [0, 1, 8], 7: [0, 8], 8: [2, 3, 5, 8]}
_COL_RANGES = {0: 4, 1: 6, 2: 6, 3: 8, 4: 15, 5: 2, 8: 9}

_K = 256
_HALF = 128
_BLOCK = 3200


def _span_of(t, f):
    feats = _EDGE_DISC_FEATS[t]
    nd = len(feats)
    per, rem = _EMB_DIM // nd, _EMB_DIM % nd
    col = 0
    for i, ff in enumerate(feats):
        dim = per + (1 if i < rem else 0)
        if ff == f:
            return col, dim
        col += dim
    raise KeyError((t, f))


_CONT_LANES = []
for _t in range(9):
    for _c in _EDGE_CONT[_t]:
        _CONT_LANES.append((_c, _t))
_NC = len(_CONT_LANES)

_DISC_LANES = []
_PLACE = []
for _c in [0, 1, 2, 3, 4, 5]:
    for _t in range(9):
        if _c in _EDGE_DISC_FEATS[_t]:
            _lo, _dim = _span_of(_t, _c)
            _PLACE.append((_NC + len(_DISC_LANES), _t, _c, 0,
                           _COL_RANGES[_c], _lo, _dim))
            for _v in range(_COL_RANGES[_c]):
                _DISC_LANES.append((_c, _t, _v))
_C8_LANE = _NC + len(_DISC_LANES)
for _v in range(9):
    _lo, _dim = _span_of(_v, 8)
    _PLACE.append((_NC + len(_DISC_LANES), _v, 8, _v, 1, _lo, _dim))
    _DISC_LANES.append((8, _v, _v))
_ND = len(_DISC_LANES)
assert _NC + _ND <= _K and _NC <= _HALF

_SS = np.zeros((16, _K + _HALF), np.float32)
_SS[14, :_K] = -1.0
for _i, (_c, _t) in enumerate(_CONT_LANES):
    _SS[8, _i] = 16.0
    _SS[14, _i] = -16.0 * _t
    _SS[_c, _K + _i] = 1.0
for _j, (_c, _t, _v) in enumerate(_DISC_LANES):
    _L = _NC + _j
    _SS[_c, _L] = 1.0 + (16.0 if _c == 8 else 0.0)
    if _c != 8:
        _SS[8, _L] = 16.0
    _SS[14, _L] = -(_v + 16.0 * _t)
    if _L < _HALF:
        _SS[14, _K + _L] = 1.0


def _body(a_ref, ss_ref, wt_ref, bias_ref, *rest):
    tbl_refs = rest[:len(_PLACE)]
    o_ref, g_ref = rest[len(_PLACE)], rest[len(_PLACE) + 1]

    @pl.when(pl.program_id(0) == 0)
    def _assemble():
        g_ref[:, :] = jnp.zeros((_K, _EMB_DIM), jnp.bfloat16)
        g_ref[0:96, :] = wt_ref[:, :].T.astype(jnp.bfloat16)
        for (lane, t, c, srow, rows, lo, dim), ref in zip(_PLACE, tbl_refs):
            g_ref[lane:lane + rows, lo:lo + dim] = (
                ref[srow:srow + rows, :].astype(jnp.bfloat16))
        g_ref[_C8_LANE:_C8_LANE + 9, :] = (
            g_ref[_C8_LANE:_C8_LANE + 9, :]
            + bias_ref[:, :].astype(jnp.bfloat16))

    a = a_ref[:, :]
    b = a.shape[0]
    az = jnp.concatenate(
        [a, jnp.ones((b, 1), jnp.float32), jnp.zeros((b, 1), jnp.float32)],
        axis=1).astype(jnp.bfloat16)
    mm = jnp.dot(az, ss_ref[:, :], preferred_element_type=jnp.float32)
    hit = mm[:, :_K] == 0.0
    lo = jnp.where(hit[:, :_HALF], mm[:, _K:], 0.0).astype(jnp.bfloat16)
    hi = hit[:, _HALF:].astype(jnp.bfloat16)
    phi = jnp.concatenate([lo, hi], axis=1)
    o_ref[:, :] = jnp.dot(phi, g_ref[:, :],
                          preferred_element_type=jnp.float32)


def kernel(edge_attr, params):
    n = edge_attr.shape[0]
    grid = n // _BLOCK

    wt = jnp.concatenate(
        [params["W"][str(t)] for t in range(9)], axis=1)
    bias = jnp.stack([params["b"][str(t)] for t in range(9)])

    tbls = [params["tables"][str(t)][str(c)]
            for (_, t, c, _, _, _, _) in _PLACE]
    in_specs = [
        pl.BlockSpec((_BLOCK, 14), lambda i: (i, 0)),
        pl.BlockSpec((16, _K + _HALF), lambda i: (0, 0)),
        pl.BlockSpec((_EMB_DIM, 96), lambda i: (0, 0)),
        pl.BlockSpec((9, _EMB_DIM), lambda i: (0, 0)),
    ] + [pl.BlockSpec(t.shape, lambda i: (0, 0)) for t in tbls]
    return pl.pallas_call(
        _body,
        grid=(grid,),
        in_specs=in_specs,
        out_specs=pl.BlockSpec((_BLOCK, _EMB_DIM), lambda i: (i, 0)),
        out_shape=jax.ShapeDtypeStruct((n, _EMB_DIM), jnp.float32),
        scratch_shapes=[pltpu.VMEM((_K, _EMB_DIM), jnp.bfloat16)],
    )(edge_attr, jnp.asarray(_SS, jnp.bfloat16), wt, bias, *tbls)


# speedup vs baseline: 1.3005x; 1.0307x over previous
import numpy as np
import jax
import jax.numpy as jnp
from jax.experimental import pallas as pl
from jax.experimental.pallas import tpu as pltpu

_EMB_DIM = 128
_EDGE_CONT = {0: [3, 6, 7, 9, 10, 11, 12, 13], 1: [2, 3, 4, 5, 6, 7, 9, 10, 11, 12, 13], 2: [2, 3, 4, 5, 6, 7, 9, 10, 11, 12, 13], 3: [1, 4, 5, 6, 7, 9, 10, 11, 12, 13], 4: [2, 3, 4, 5, 6, 7, 9, 10, 11, 12, 13], 5: [1, 2, 3, 4, 5, 6, 7, 9, 10, 11, 12, 13], 6: [2, 3, 4, 5, 6, 7, 9, 10, 11, 12, 13], 7: [1, 2, 3, 4, 5, 6, 7, 9, 10, 11, 12, 13], 8: [0, 1, 4, 6, 7, 9, 10, 11, 12, 13]}
_EDGE_DISC_FEATS = {0: [0, 1, 2, 4, 5, 8], 1: [0, 1, 8], 2: [0, 1, 8], 3: [0, 2, 3, 8], 4: [0, 1, 8], 5: [0, 8], 6: [0, 1, 8], 7: [0, 8], 8: [2, 3, 5, 8]}
_COL_RANGES = {0: 4, 1: 6, 2: 6, 3: 8, 4: 15, 5: 2, 8: 9}

_K = 256
_HALF = 128
_BLOCK = 6400


def _span_of(t, f):
    feats = _EDGE_DISC_FEATS[t]
    nd = len(feats)
    per, rem = _EMB_DIM // nd, _EMB_DIM % nd
    col = 0
    for i, ff in enumerate(feats):
        dim = per + (1 if i < rem else 0)
        if ff == f:
            return col, dim
        col += dim
    raise KeyError((t, f))


_CONT_LANES = []
for _t in range(9):
    for _c in _EDGE_CONT[_t]:
        _CONT_LANES.append((_c, _t))
_NC = len(_CONT_LANES)

_DISC_LANES = []
_PLACE = []
for _c in [0, 1, 2, 3, 4, 5]:
    for _t in range(9):
        if _c in _EDGE_DISC_FEATS[_t]:
            _lo, _dim = _span_of(_t, _c)
            _PLACE.append((_NC + len(_DISC_LANES), _t, _c, 0,
                           _COL_RANGES[_c], _lo, _dim))
            for _v in range(_COL_RANGES[_c]):
                _DISC_LANES.append((_c, _t, _v))
_C8_LANE = _NC + len(_DISC_LANES)
for _v in range(9):
    _lo, _dim = _span_of(_v, 8)
    _PLACE.append((_NC + len(_DISC_LANES), _v, 8, _v, 1, _lo, _dim))
    _DISC_LANES.append((8, _v, _v))
_ND = len(_DISC_LANES)
assert _NC + _ND <= _K and _NC <= _HALF

_SS = np.zeros((16, _K + _HALF), np.float32)
_SS[14, :_K] = -1.0
for _i, (_c, _t) in enumerate(_CONT_LANES):
    _SS[8, _i] = 16.0
    _SS[14, _i] = -16.0 * _t
    _SS[_c, _K + _i] = 1.0
for _j, (_c, _t, _v) in enumerate(_DISC_LANES):
    _L = _NC + _j
    _SS[_c, _L] = 1.0 + (16.0 if _c == 8 else 0.0)
    if _c != 8:
        _SS[8, _L] = 16.0
    _SS[14, _L] = -(_v + 16.0 * _t)
    if _L < _HALF:
        _SS[14, _K + _L] = 1.0


def _body(a_ref, ss_ref, wt_ref, bias_ref, *rest):
    tbl_refs = rest[:len(_PLACE)]
    o_ref, g_ref = rest[len(_PLACE)], rest[len(_PLACE) + 1]

    @pl.when(pl.program_id(0) == 0)
    def _assemble():
        g_ref[:, :] = jnp.zeros((_K, _EMB_DIM), jnp.bfloat16)
        g_ref[0:96, :] = wt_ref[:, :].T.astype(jnp.bfloat16)
        for (lane, t, c, srow, rows, lo, dim), ref in zip(_PLACE, tbl_refs):
            g_ref[lane:lane + rows, lo:lo + dim] = (
                ref[srow:srow + rows, :].astype(jnp.bfloat16))
        g_ref[_C8_LANE:_C8_LANE + 9, :] = (
            g_ref[_C8_LANE:_C8_LANE + 9, :]
            + bias_ref[:, :].astype(jnp.bfloat16))

    a = a_ref[:, :]
    b = a.shape[0]
    az = jnp.concatenate(
        [a, jnp.ones((b, 1), jnp.float32), jnp.zeros((b, 1), jnp.float32)],
        axis=1).astype(jnp.bfloat16)
    mm = jnp.dot(az, ss_ref[:, :], preferred_element_type=jnp.float32)
    hit = mm[:, :_K] == 0.0
    lo = jnp.where(hit[:, :_HALF], mm[:, _K:], 0.0).astype(jnp.bfloat16)
    hi = hit[:, _HALF:].astype(jnp.bfloat16)
    phi = jnp.concatenate([lo, hi], axis=1)
    o_ref[:, :] = jnp.dot(phi, g_ref[:, :],
                          preferred_element_type=jnp.float32)


def kernel(edge_attr, params):
    n = edge_attr.shape[0]
    grid = n // _BLOCK

    wt = jnp.concatenate(
        [params["W"][str(t)] for t in range(9)], axis=1)
    bias = jnp.stack([params["b"][str(t)] for t in range(9)])

    tbls = [params["tables"][str(t)][str(c)]
            for (_, t, c, _, _, _, _) in _PLACE]
    in_specs = [
        pl.BlockSpec((_BLOCK, 14), lambda i: (i, 0)),
        pl.BlockSpec((16, _K + _HALF), lambda i: (0, 0)),
        pl.BlockSpec((_EMB_DIM, 96), lambda i: (0, 0)),
        pl.BlockSpec((9, _EMB_DIM), lambda i: (0, 0)),
    ] + [pl.BlockSpec(t.shape, lambda i: (0, 0)) for t in tbls]
    return pl.pallas_call(
        _body,
        grid=(grid,),
        in_specs=in_specs,
        out_specs=pl.BlockSpec((_BLOCK, _EMB_DIM), lambda i: (i, 0)),
        out_shape=jax.ShapeDtypeStruct((n, _EMB_DIM), jnp.float32),
        scratch_shapes=[pltpu.VMEM((_K, _EMB_DIM), jnp.bfloat16)],
    )(edge_attr, jnp.asarray(_SS, jnp.bfloat16), wt, bias, *tbls)


# speedup vs baseline: 1.5219x; 1.1702x over previous
import numpy as np
import jax
import jax.numpy as jnp
from jax.experimental import pallas as pl
from jax.experimental.pallas import tpu as pltpu

_EMB_DIM = 128
_EDGE_CONT = {0: [3, 6, 7, 9, 10, 11, 12, 13], 1: [2, 3, 4, 5, 6, 7, 9, 10, 11, 12, 13], 2: [2, 3, 4, 5, 6, 7, 9, 10, 11, 12, 13], 3: [1, 4, 5, 6, 7, 9, 10, 11, 12, 13], 4: [2, 3, 4, 5, 6, 7, 9, 10, 11, 12, 13], 5: [1, 2, 3, 4, 5, 6, 7, 9, 10, 11, 12, 13], 6: [2, 3, 4, 5, 6, 7, 9, 10, 11, 12, 13], 7: [1, 2, 3, 4, 5, 6, 7, 9, 10, 11, 12, 13], 8: [0, 1, 4, 6, 7, 9, 10, 11, 12, 13]}
_EDGE_DISC_FEATS = {0: [0, 1, 2, 4, 5, 8], 1: [0, 1, 8], 2: [0, 1, 8], 3: [0, 2, 3, 8], 4: [0, 1, 8], 5: [0, 8], 6: [0, 1, 8], 7: [0, 8], 8: [2, 3, 5, 8]}
_COL_RANGES = {0: 4, 1: 6, 2: 6, 3: 8, 4: 15, 5: 2, 8: 9}

_K = 256
_HALF = 128
_BLOCK = 10000


def _span_of(t, f):
    feats = _EDGE_DISC_FEATS[t]
    nd = len(feats)
    per, rem = _EMB_DIM // nd, _EMB_DIM % nd
    col = 0
    for i, ff in enumerate(feats):
        dim = per + (1 if i < rem else 0)
        if ff == f:
            return col, dim
        col += dim
    raise KeyError((t, f))


_CONT_LANES = []
for _t in range(9):
    for _c in _EDGE_CONT[_t]:
        _CONT_LANES.append((_c, _t))
_NC = len(_CONT_LANES)

_DISC_LANES = []
_PLACE = []
for _c in [0, 1, 2, 3, 4, 5]:
    for _t in range(9):
        if _c in _EDGE_DISC_FEATS[_t]:
            _lo, _dim = _span_of(_t, _c)
            _PLACE.append((_NC + len(_DISC_LANES), _t, _c, 0,
                           _COL_RANGES[_c], _lo, _dim))
            for _v in range(_COL_RANGES[_c]):
                _DISC_LANES.append((_c, _t, _v))
_C8_LANE = _NC + len(_DISC_LANES)
for _v in range(9):
    _lo, _dim = _span_of(_v, 8)
    _PLACE.append((_NC + len(_DISC_LANES), _v, 8, _v, 1, _lo, _dim))
    _DISC_LANES.append((8, _v, _v))
_ND = len(_DISC_LANES)
assert _NC + _ND <= _K and _NC <= _HALF

_SS = np.zeros((16, _K + _HALF), np.float32)
_SS[14, :_K] = -1.0
for _i, (_c, _t) in enumerate(_CONT_LANES):
    _SS[8, _i] = 16.0
    _SS[14, _i] = -16.0 * _t
    _SS[_c, _K + _i] = 1.0
for _j, (_c, _t, _v) in enumerate(_DISC_LANES):
    _L = _NC + _j
    _SS[_c, _L] = 1.0 + (16.0 if _c == 8 else 0.0)
    if _c != 8:
        _SS[8, _L] = 16.0
    _SS[14, _L] = -(_v + 16.0 * _t)
    if _L < _HALF:
        _SS[14, _K + _L] = 1.0


def _body(a_ref, ss_ref, wt_ref, bias_ref, *rest):
    tbl_refs = rest[:len(_PLACE)]
    o_ref, g_ref = rest[len(_PLACE)], rest[len(_PLACE) + 1]

    @pl.when(pl.program_id(0) == 0)
    def _assemble():
        g_ref[:, :] = jnp.zeros((_K, _EMB_DIM), jnp.bfloat16)
        g_ref[0:96, :] = wt_ref[:, :].T.astype(jnp.bfloat16)
        for (lane, t, c, srow, rows, lo, dim), ref in zip(_PLACE, tbl_refs):
            g_ref[lane:lane + rows, lo:lo + dim] = (
                ref[srow:srow + rows, :].astype(jnp.bfloat16))
        g_ref[_C8_LANE:_C8_LANE + 9, :] = (
            g_ref[_C8_LANE:_C8_LANE + 9, :]
            + bias_ref[:, :].astype(jnp.bfloat16))

    a = a_ref[:, :]
    b = a.shape[0]
    az = jnp.concatenate(
        [a, jnp.ones((b, 1), jnp.float32), jnp.zeros((b, 1), jnp.float32)],
        axis=1).astype(jnp.bfloat16)
    mm = jnp.dot(az, ss_ref[:, :], preferred_element_type=jnp.float32)
    hit = mm[:, :_K] == 0.0
    lo = jnp.where(hit[:, :_HALF], mm[:, _K:], 0.0).astype(jnp.bfloat16)
    hi = hit[:, _HALF:].astype(jnp.bfloat16)
    phi = jnp.concatenate([lo, hi], axis=1)
    o_ref[:, :] = jnp.dot(phi, g_ref[:, :],
                          preferred_element_type=jnp.float32)


def kernel(edge_attr, params):
    n = edge_attr.shape[0]
    grid = n // _BLOCK

    wt = jnp.concatenate(
        [params["W"][str(t)] for t in range(9)], axis=1)
    bias = jnp.stack([params["b"][str(t)] for t in range(9)])

    tbls = [params["tables"][str(t)][str(c)]
            for (_, t, c, _, _, _, _) in _PLACE]
    in_specs = [
        pl.BlockSpec((_BLOCK, 14), lambda i: (i, 0)),
        pl.BlockSpec((16, _K + _HALF), lambda i: (0, 0)),
        pl.BlockSpec((_EMB_DIM, 96), lambda i: (0, 0)),
        pl.BlockSpec((9, _EMB_DIM), lambda i: (0, 0)),
    ] + [pl.BlockSpec(t.shape, lambda i: (0, 0)) for t in tbls]
    return pl.pallas_call(
        _body,
        grid=(grid,),
        in_specs=in_specs,
        out_specs=pl.BlockSpec((_BLOCK, _EMB_DIM), lambda i: (i, 0)),
        out_shape=jax.ShapeDtypeStruct((n, _EMB_DIM), jnp.float32),
        scratch_shapes=[pltpu.VMEM((_K, _EMB_DIM), jnp.bfloat16)],
    )(edge_attr, jnp.asarray(_SS, jnp.bfloat16), wt, bias, *tbls)
